# Initial kernel scaffold; baseline (speedup 1.0000x reference)
import jax
import jax.numpy as jnp
from jax.experimental import pallas as pl


def kernel(word_embeddings, texts, common_tbl, demo_tbl, rep_tbl, W, b):
    raise NotImplementedError("write your pallas kernel here")


# speedup vs baseline: 7.4964x; 7.4964x over previous
import functools

import jax
import jax.numpy as jnp
from jax import lax
from jax.experimental import pallas as pl
from jax.experimental.pallas import tpu as pltpu
from jax.experimental.pallas import tpu_sc as plsc

VOCAB = 100000
EMBED = 128

_TBL_BLK = 2000
_PROJ_BLK = 2048
_GATHER_CHUNK = 128


def _fuse_tables_body(c_ref, d_ref, r_ref, w_ref, b_ref, t_ref):
    w = w_ref[...]
    w1 = w[:, :EMBED]
    w2 = w[:, EMBED:]
    dn = (((1,), (1,)), ((), ()))
    acc = lax.dot_general(c_ref[...], (w1 + w2) * 0.25, dn,
                          preferred_element_type=jnp.float32)
    acc += lax.dot_general(d_ref[...], w1 * 0.5, dn,
                           preferred_element_type=jnp.float32)
    acc += lax.dot_general(r_ref[...], w2 * 0.5, dn,
                           preferred_element_type=jnp.float32)
    t_ref[...] = acc + b_ref[...]


def _fuse_tables(c, d, r, w, b2d):
    n_blk = VOCAB // _TBL_BLK
    tbl_spec = pl.BlockSpec((_TBL_BLK, EMBED), lambda i: (i, 0))
    return pl.pallas_call(
        _fuse_tables_body,
        grid=(n_blk,),
        in_specs=[tbl_spec, tbl_spec, tbl_spec,
                  pl.BlockSpec((EMBED, 2 * EMBED), lambda i: (0, 0)),
                  pl.BlockSpec((1, EMBED), lambda i: (0, 0))],
        out_specs=tbl_spec,
        out_shape=jax.ShapeDtypeStruct((VOCAB, EMBED), jnp.float32),
    )(c, d, r, w, b2d)


def _proj_add_body(x_ref, g_ref, w_ref, o_ref):
    w = w_ref[...]
    ws = (w[:, :EMBED] + w[:, EMBED:]) * 0.25
    dn = (((1,), (1,)), ((), ()))
    o_ref[...] = lax.dot_general(x_ref[...], ws, dn,
                                 preferred_element_type=jnp.float32) + g_ref[...]


def _proj_add(x, g, w):
    n = x.shape[0]
    n_blk = n // _PROJ_BLK
    row_spec = pl.BlockSpec((_PROJ_BLK, EMBED), lambda i: (i, 0))
    return pl.pallas_call(
        _proj_add_body,
        grid=(n_blk,),
        in_specs=[row_spec, row_spec,
                  pl.BlockSpec((EMBED, 2 * EMBED), lambda i: (0, 0))],
        out_specs=row_spec,
        out_shape=jax.ShapeDtypeStruct((n, EMBED), jnp.float32),
    )(x, g, w)


@functools.cache
def _make_gather(n_rows):
    info = plsc.get_sparse_core_info()
    nc, ns = info.num_cores, info.num_subcores
    nw = nc * ns
    rows_per_w = n_rows // nw
    chunks = rows_per_w // _GATHER_CHUNK
    mesh = plsc.VectorSubcoreMesh(core_axis_name="c", subcore_axis_name="s")

    @functools.partial(
        pl.kernel,
        mesh=mesh,
        out_type=jax.ShapeDtypeStruct((n_rows, EMBED), jnp.float32),
        scratch_types=[
            pltpu.VMEM((chunks, _GATHER_CHUNK), jnp.int32),
            pltpu.VMEM((_GATHER_CHUNK, EMBED), jnp.float32),
            pltpu.VMEM((_GATHER_CHUNK, EMBED), jnp.float32),
            pltpu.SemaphoreType.DMA,
            pltpu.SemaphoreType.DMA,
        ],
    )
    def gather(t_hbm, idx_hbm, out_hbm, idx_v, rows0, rows1, sem0, sem1):
        wid = lax.axis_index("s") * nc + lax.axis_index("c")
        pltpu.sync_copy(idx_hbm.at[wid], idx_v)
        base = wid * chunks * _GATHER_CHUNK

        pltpu.async_copy(t_hbm.at[idx_v.at[0]], rows0, sem0)

        def step(p, carry):
            j0 = p * 2
            pltpu.make_async_copy(t_hbm.at[idx_v.at[j0]], rows0, sem0).wait()
            pltpu.async_copy(t_hbm.at[idx_v.at[j0 + 1]], rows1, sem1)
            pltpu.sync_copy(
                rows0,
                out_hbm.at[pl.ds(base + j0 * _GATHER_CHUNK, _GATHER_CHUNK)])
            pltpu.make_async_copy(t_hbm.at[idx_v.at[j0 + 1]], rows1,
                                  sem1).wait()

            @pl.when(j0 + 2 < chunks)
            def _prefetch():
                pltpu.async_copy(t_hbm.at[idx_v.at[j0 + 2]], rows0, sem0)

            pltpu.sync_copy(
                rows1,
                out_hbm.at[pl.ds(base + (j0 + 1) * _GATHER_CHUNK,
                                 _GATHER_CHUNK)])
            return carry

        lax.fori_loop(0, chunks // 2, step, 0)

    return gather


def kernel(word_embeddings, texts, common_tbl, demo_tbl, rep_tbl, W, b):
    bsz, seq, emb = word_embeddings.shape
    n = bsz * seq
    info = plsc.get_sparse_core_info()
    nw = info.num_cores * info.num_subcores
    idx3d = texts.reshape(nw, n // (nw * _GATHER_CHUNK),
                          _GATHER_CHUNK).astype(jnp.int32)
    fused_tbl = _fuse_tables(common_tbl, demo_tbl, rep_tbl, W,
                             b.reshape(1, emb))
    gathered = _make_gather(n)(fused_tbl, idx3d)
    out = _proj_add(word_embeddings.reshape(n, emb), gathered, W)
    return out.reshape(bsz, seq, emb)


# speedup vs baseline: 8.0373x; 1.0722x over previous
import functools

import jax
import jax.numpy as jnp
from jax import lax
from jax.experimental import pallas as pl
from jax.experimental.pallas import tpu as pltpu
from jax.experimental.pallas import tpu_sc as plsc

VOCAB = 100000
EMBED = 128

_TBL_BLK = 2000
_PROJ_BLK = 2048
_GATHER_CHUNK = 128
_K = 5


def _fuse_tables_body(c_ref, d_ref, r_ref, w_ref, b_ref, t_ref):
    w = w_ref[...]
    w1 = w[:, :EMBED]
    w2 = w[:, EMBED:]
    dn = (((1,), (1,)), ((), ()))
    acc = lax.dot_general(c_ref[...], (w1 + w2) * 0.25, dn,
                          preferred_element_type=jnp.float32)
    acc += lax.dot_general(d_ref[...], w1 * 0.5, dn,
                           preferred_element_type=jnp.float32)
    acc += lax.dot_general(r_ref[...], w2 * 0.5, dn,
                           preferred_element_type=jnp.float32)
    t_ref[...] = acc + b_ref[...]


def _fuse_tables(c, d, r, w, b2d):
    n_blk = VOCAB // _TBL_BLK
    tbl_spec = pl.BlockSpec((_TBL_BLK, EMBED), lambda i: (i, 0))
    return pl.pallas_call(
        _fuse_tables_body,
        grid=(n_blk,),
        in_specs=[tbl_spec, tbl_spec, tbl_spec,
                  pl.BlockSpec((EMBED, 2 * EMBED), lambda i: (0, 0)),
                  pl.BlockSpec((1, EMBED), lambda i: (0, 0))],
        out_specs=tbl_spec,
        out_shape=jax.ShapeDtypeStruct((VOCAB, EMBED), jnp.float32),
        name="fuse_tables",
    )(c, d, r, w, b2d)


def _proj_add_body(x_ref, g_ref, w_ref, o_ref):
    w = w_ref[...]
    ws = (w[:, :EMBED] + w[:, EMBED:]) * 0.25
    dn = (((1,), (1,)), ((), ()))
    o_ref[...] = lax.dot_general(x_ref[...], ws, dn,
                                 preferred_element_type=jnp.float32) + g_ref[...]


def _proj_add_chunk(ci, x, g, w, prev_out):
    n = x.shape[0]
    steps = n // _K // _PROJ_BLK
    off = ci * steps
    row_spec = pl.BlockSpec((_PROJ_BLK, EMBED), lambda i: (off + i, 0))
    operands = [x, g, w]
    in_specs = [row_spec, row_spec,
                pl.BlockSpec((EMBED, 2 * EMBED), lambda i: (0, 0))]
    aliases = {}
    if prev_out is not None:
        operands.append(prev_out)
        in_specs.append(pl.BlockSpec(memory_space=pl.ANY))
        aliases = {3: 0}

    def body(x_ref, g_ref, w_ref, *rest):
        _proj_add_body(x_ref, g_ref, w_ref, rest[-1])

    return pl.pallas_call(
        body,
        grid=(steps,),
        in_specs=in_specs,
        out_specs=row_spec,
        out_shape=jax.ShapeDtypeStruct((n, EMBED), jnp.float32),
        input_output_aliases=aliases,
        name=f"proj_add_{ci}",
    )(*operands)


@functools.cache
def _make_gather(n_rows, ci):
    info = plsc.get_sparse_core_info()
    nc, ns = info.num_cores, info.num_subcores
    nw = nc * ns
    chunks = n_rows // _K // nw // _GATHER_CHUNK
    mesh = plsc.VectorSubcoreMesh(core_axis_name="c", subcore_axis_name="s")

    @functools.partial(
        pl.kernel,
        mesh=mesh,
        out_type=jax.ShapeDtypeStruct((n_rows, EMBED), jnp.float32),
        scratch_types=[
            pltpu.VMEM((chunks, _GATHER_CHUNK), jnp.int32),
            pltpu.VMEM((_GATHER_CHUNK, EMBED), jnp.float32),
            pltpu.VMEM((_GATHER_CHUNK, EMBED), jnp.float32),
            pltpu.SemaphoreType.DMA,
            pltpu.SemaphoreType.DMA,
        ],
        name=f"sc_gather_{ci}",
    )
    def gather(t_hbm, idx_hbm, out_hbm, idx_v, rows0, rows1, sem0, sem1):
        wid = lax.axis_index("s") * nc + lax.axis_index("c")
        pltpu.sync_copy(idx_hbm.at[ci * nw + wid], idx_v)
        base = (ci * nw + wid) * chunks * _GATHER_CHUNK

        pltpu.async_copy(t_hbm.at[idx_v.at[0]], rows0, sem0)

        def step(p, carry):
            j0 = p * 2
            pltpu.make_async_copy(t_hbm.at[idx_v.at[j0]], rows0, sem0).wait()
            pltpu.async_copy(t_hbm.at[idx_v.at[j0 + 1]], rows1, sem1)
            pltpu.sync_copy(
                rows0,
                out_hbm.at[pl.ds(base + j0 * _GATHER_CHUNK, _GATHER_CHUNK)])
            pltpu.make_async_copy(t_hbm.at[idx_v.at[j0 + 1]], rows1,
                                  sem1).wait()

            @pl.when(j0 + 2 < chunks)
            def _prefetch():
                pltpu.async_copy(t_hbm.at[idx_v.at[j0 + 2]], rows0, sem0)

            pltpu.sync_copy(
                rows1,
                out_hbm.at[pl.ds(base + (j0 + 1) * _GATHER_CHUNK,
                                 _GATHER_CHUNK)])
            return carry

        lax.fori_loop(0, chunks // 2, step, 0)

    return gather


def kernel(word_embeddings, texts, common_tbl, demo_tbl, rep_tbl, W, b):
    bsz, seq, emb = word_embeddings.shape
    n = bsz * seq
    info = plsc.get_sparse_core_info()
    nw = info.num_cores * info.num_subcores
    idx3d = texts.reshape(_K * nw, n // (_K * nw * _GATHER_CHUNK),
                          _GATHER_CHUNK).astype(jnp.int32)
    fused_tbl = _fuse_tables(common_tbl, demo_tbl, rep_tbl, W,
                             b.reshape(1, emb))
    gathered = [_make_gather(n, ci)(fused_tbl, idx3d) for ci in range(_K)]
    we_flat = word_embeddings.reshape(n, emb)
    out = None
    for ci in range(_K):
        out = _proj_add_chunk(ci, we_flat, gathered[ci], W, out)
    return out.reshape(bsz, seq, emb)


# speedup vs baseline: 8.7238x; 1.0854x over previous
import functools

import jax
import jax.numpy as jnp
from jax import lax
from jax.experimental import pallas as pl
from jax.experimental.pallas import tpu as pltpu
from jax.experimental.pallas import tpu_sc as plsc

VOCAB = 100000
EMBED = 128

_TBL_BLK = 4000
_PROJ_BLK = 4096
_GATHER_CHUNK = 128
_K = 5


def _fuse_tables_body(c_ref, d_ref, r_ref, w_ref, b_ref, t_ref):
    w = w_ref[...]
    w1 = w[:, :EMBED]
    w2 = w[:, EMBED:]
    dn = (((1,), (1,)), ((), ()))
    acc = lax.dot_general(c_ref[...], (w1 + w2) * 0.25, dn,
                          preferred_element_type=jnp.float32)
    acc += lax.dot_general(d_ref[...], w1 * 0.5, dn,
                           preferred_element_type=jnp.float32)
    acc += lax.dot_general(r_ref[...], w2 * 0.5, dn,
                           preferred_element_type=jnp.float32)
    t_ref[...] = acc + b_ref[...]


def _fuse_tables(c, d, r, w, b2d):
    n_blk = VOCAB // _TBL_BLK
    tbl_spec = pl.BlockSpec((_TBL_BLK, EMBED), lambda i: (i, 0))
    return pl.pallas_call(
        _fuse_tables_body,
        grid=(n_blk,),
        in_specs=[tbl_spec, tbl_spec, tbl_spec,
                  pl.BlockSpec((EMBED, 2 * EMBED), lambda i: (0, 0)),
                  pl.BlockSpec((1, EMBED), lambda i: (0, 0))],
        out_specs=tbl_spec,
        out_shape=jax.ShapeDtypeStruct((VOCAB, EMBED), jnp.float32),
        name="fuse_tables",
    )(c, d, r, w, b2d)


def _proj_add_body(x_ref, g_ref, w_ref, o_ref):
    w = w_ref[...]
    ws = (w[:, :EMBED] + w[:, EMBED:]) * 0.25
    dn = (((1,), (1,)), ((), ()))
    o_ref[...] = lax.dot_general(x_ref[...], ws, dn,
                                 preferred_element_type=jnp.float32
                                 ) + g_ref[...]


def _proj_add_chunk(ci, x, g, w, prev_out):
    n = x.shape[0]
    steps = n // _K // _PROJ_BLK
    off = ci * steps
    row_spec = pl.BlockSpec((_PROJ_BLK, EMBED), lambda i: (off + i, 0))
    operands = [x, g, w]
    in_specs = [row_spec, row_spec,
                pl.BlockSpec((EMBED, 2 * EMBED), lambda i: (0, 0))]
    aliases = {}
    if prev_out is not None:
        operands.append(prev_out)
        in_specs.append(pl.BlockSpec(memory_space=pl.ANY))
        aliases = {3: 0}

    def body(x_ref, g_ref, w_ref, *rest):
        _proj_add_body(x_ref, g_ref, w_ref, rest[-1])

    return pl.pallas_call(
        body,
        grid=(steps,),
        in_specs=in_specs,
        out_specs=row_spec,
        out_shape=jax.ShapeDtypeStruct((n, EMBED), jnp.float32),
        input_output_aliases=aliases,
        name=f"proj_add_{ci}",
    )(*operands)


@functools.cache
def _make_gather(n_rows, ci):
    info = plsc.get_sparse_core_info()
    nc, ns = info.num_cores, info.num_subcores
    nw = nc * ns
    chunks = n_rows // _K // nw // _GATHER_CHUNK
    mesh = plsc.VectorSubcoreMesh(core_axis_name="c", subcore_axis_name="s")

    @functools.partial(
        pl.kernel,
        mesh=mesh,
        out_type=jax.ShapeDtypeStruct((n_rows, EMBED), jnp.float32),
        scratch_types=[
            pltpu.VMEM((chunks, _GATHER_CHUNK), jnp.int32),
            pltpu.VMEM((_GATHER_CHUNK, EMBED), jnp.float32),
            pltpu.VMEM((_GATHER_CHUNK, EMBED), jnp.float32),
            pltpu.SemaphoreType.DMA,
            pltpu.SemaphoreType.DMA,
        ],
        name=f"sc_gather_{ci}",
    )
    def gather(t_hbm, idx_hbm, out_hbm, idx_v, rows0, rows1, sem0, sem1):
        wid = lax.axis_index("s") * nc + lax.axis_index("c")
        pltpu.sync_copy(idx_hbm.at[ci * nw + wid], idx_v)
        base = (ci * nw + wid) * chunks * _GATHER_CHUNK

        pltpu.async_copy(t_hbm.at[idx_v.at[0]], rows0, sem0)

        def step(p, carry):
            j0 = p * 2
            pltpu.make_async_copy(t_hbm.at[idx_v.at[j0]], rows0, sem0).wait()
            pltpu.async_copy(t_hbm.at[idx_v.at[j0 + 1]], rows1, sem1)
            pltpu.sync_copy(
                rows0,
                out_hbm.at[pl.ds(base + j0 * _GATHER_CHUNK, _GATHER_CHUNK)])
            pltpu.make_async_copy(t_hbm.at[idx_v.at[j0 + 1]], rows1,
                                  sem1).wait()

            @pl.when(j0 + 2 < chunks)
            def _prefetch():
                pltpu.async_copy(t_hbm.at[idx_v.at[j0 + 2]], rows0, sem0)

            pltpu.sync_copy(
                rows1,
                out_hbm.at[pl.ds(base + (j0 + 1) * _GATHER_CHUNK,
                                 _GATHER_CHUNK)])
            return carry

        lax.fori_loop(0, chunks // 2, step, 0)

    return gather


def kernel(word_embeddings, texts, common_tbl, demo_tbl, rep_tbl, W, b):
    bsz, seq, emb = word_embeddings.shape
    n = bsz * seq
    info = plsc.get_sparse_core_info()
    nw = info.num_cores * info.num_subcores
    idx3d = texts.reshape(_K * nw, n // (_K * nw * _GATHER_CHUNK),
                          _GATHER_CHUNK).astype(jnp.int32)
    fused_tbl = _fuse_tables(common_tbl, demo_tbl, rep_tbl, W,
                             b.reshape(1, emb))
    gathered = [_make_gather(n, ci)(fused_tbl, idx3d) for ci in range(_K)]
    we_flat = word_embeddings.reshape(n, emb)
    out = None
    for ci in range(_K):
        out = _proj_add_chunk(ci, we_flat, gathered[ci], W, out)
    return out.reshape(bsz, seq, emb)
